# baseline (speedup 1.0000x reference)
import jax
import jax.numpy as jnp
from jax.experimental import pallas as pl
from jax.experimental.pallas import tpu as pltpu

_BLOCK_SIZE = 8192
_N_EMBD = 1024
_NSPLIT = 4
_ROWS = _BLOCK_SIZE // _NSPLIT


def _dma_body(x_ref, o_ref, *sems):
    copies = [
        pltpu.make_async_copy(
            x_ref.at[pl.ds(i * _ROWS, _ROWS)],
            o_ref.at[pl.ds(i * _ROWS, _ROWS)],
            sems[i],
        )
        for i in range(_NSPLIT)
    ]
    for c in copies:
        c.start()
    for c in copies:
        c.wait()


def kernel(wpe):
    out = pl.pallas_call(
        _dma_body,
        in_specs=[pl.BlockSpec(memory_space=pl.ANY)],
        out_specs=pl.BlockSpec(memory_space=pl.ANY),
        out_shape=jax.ShapeDtypeStruct((_BLOCK_SIZE, _N_EMBD), jnp.float32),
        scratch_shapes=[pltpu.SemaphoreType.DMA] * _NSPLIT,
    )(wpe)
    return out[None]


# speedup vs baseline: 44.5376x; 44.5376x over previous
import jax
import jax.numpy as jnp
from jax.experimental import pallas as pl

_BLOCK_SIZE = 8192
_N_EMBD = 1024
_ROWS_PER_BLOCK = 1024


def _copy_body(x_ref, o_ref):
    o_ref[...] = x_ref[...]


def kernel(wpe):
    out = pl.pallas_call(
        _copy_body,
        grid=(_BLOCK_SIZE // _ROWS_PER_BLOCK,),
        in_specs=[pl.BlockSpec((_ROWS_PER_BLOCK, _N_EMBD), lambda i: (i, 0))],
        out_specs=pl.BlockSpec((_ROWS_PER_BLOCK, _N_EMBD), lambda i: (i, 0)),
        out_shape=jax.ShapeDtypeStruct((_BLOCK_SIZE, _N_EMBD), jnp.float32),
    )(wpe)
    return out[None]


# speedup vs baseline: 47.4797x; 1.0661x over previous
import jax
import jax.numpy as jnp
from jax.experimental import pallas as pl

_BLOCK_SIZE = 8192
_N_EMBD = 1024
_ROWS_PER_BLOCK = 2048


def _copy_body(x_ref, o_ref):
    o_ref[...] = x_ref[...]


def kernel(wpe):
    out = pl.pallas_call(
        _copy_body,
        grid=(_BLOCK_SIZE // _ROWS_PER_BLOCK,),
        in_specs=[pl.BlockSpec((_ROWS_PER_BLOCK, _N_EMBD), lambda i: (i, 0))],
        out_specs=pl.BlockSpec((_ROWS_PER_BLOCK, _N_EMBD), lambda i: (i, 0)),
        out_shape=jax.ShapeDtypeStruct((_BLOCK_SIZE, _N_EMBD), jnp.float32),
    )(wpe)
    return out[None]
